# baseline (speedup 1.0000x reference)
import functools

import jax
import jax.numpy as jnp
from jax import lax
from jax.experimental import pallas as pl
from jax.experimental.pallas import tpu as pltpu
from jax.experimental.pallas import tpu_sc as plsc
import numpy as np

N_NODES = 100000
NODE_DIM = 512
HIDDEN = 256
N_GRAPHS = 512
_MEAN = 0.0
_STDDEV = 1.0
_LOG2 = float(np.log(2.0))

_SPLIT = 90000
_BLK = 10000


def _mlp_body(x_ref, w1_ref, b1_ref, w2_ref, o_ref):
    h = jnp.dot(x_ref[...], w1_ref[...], preferred_element_type=jnp.float32)
    h = h + b1_ref[...]
    sp = jnp.maximum(h, 0.0) + jnp.log1p(jnp.exp(-jnp.abs(h))) - _LOG2
    o_ref[...] = jnp.dot(sp, w2_ref[...], preferred_element_type=jnp.float32)


def _mlp(x, W1, b1, W2, row0, nrows):
    blk0 = row0 // _BLK
    return pl.pallas_call(
        _mlp_body,
        grid=(nrows // _BLK,),
        in_specs=[
            pl.BlockSpec((_BLK, NODE_DIM), lambda i: (blk0 + i, 0)),
            pl.BlockSpec((NODE_DIM, HIDDEN), lambda i: (0, 0)),
            pl.BlockSpec((1, HIDDEN), lambda i: (0, 0)),
            pl.BlockSpec((HIDDEN, 1), lambda i: (0, 0)),
        ],
        out_specs=pl.BlockSpec((_BLK, 1), lambda i: (i, 0)),
        out_shape=jax.ShapeDtypeStruct((nrows, 1), jnp.float32),
    )(x, W1, b1, W2)


_NS = 16
_COLS = N_GRAPHS // _NS


def _make_segment_sum(n, off, final):
    chunk = 16 * (n // (16 * _NS))
    tail = n - (_NS - 1) * chunk
    buf = max(chunk, tail)

    def body(*refs):
        if final:
            (vals_hbm, idx_hbm, prev_hbm, out_hbm,
             val_v, idx_v, acc2, accv, shared, gath, prev_v, sem) = refs
        else:
            (vals_hbm, idx_hbm, out_hbm,
             val_v, idx_v, acc2, accv, shared, gath, sem) = refs
        c = lax.axis_index("c")
        s = lax.axis_index("s")
        on = c == 0

        @pl.when(on)
        def _work():
            base = s * chunk

            @pl.when(s < _NS - 1)
            def _full():
                cp_v = pltpu.async_copy(
                    vals_hbm.at[pl.ds(base, chunk)], val_v.at[pl.ds(0, chunk)], sem)
                cp_i = pltpu.async_copy(
                    idx_hbm.at[pl.ds(off + base, chunk)], idx_v.at[pl.ds(0, chunk)], sem)
                cp_v.wait()
                cp_i.wait()

            @pl.when(s == _NS - 1)
            def _tail():
                cp_v = pltpu.async_copy(
                    vals_hbm.at[pl.ds(base, tail)], val_v.at[pl.ds(0, tail)], sem)
                cp_i = pltpu.async_copy(
                    idx_hbm.at[pl.ds(off + base, tail)], idx_v.at[pl.ds(0, tail)], sem)
                cp_v.wait()
                cp_i.wait()

            zero16 = jnp.zeros((16,), jnp.float32)

            def _zero(i, _):
                for k in range(8):
                    acc2[pl.ds((i * 8 + k) * 16, 16)] = zero16
                return 0

            lax.fori_loop(0, (16 * N_GRAPHS) // (16 * 8), _zero, 0)

            lane_off = lax.iota(jnp.int32, 16) * N_GRAPHS
            nvec = jnp.where(s == _NS - 1, tail // 16, chunk // 16)
            n4 = nvec // 4

            def _scat4(i, _):
                for k in range(4):
                    j = i * 4 + k
                    v = val_v[pl.ds(j * 16, 16)]
                    ix = idx_v[pl.ds(j * 16, 16)]
                    plsc.addupdate_scatter(acc2, [lane_off + ix], v)
                return 0

            lax.fori_loop(0, n4, _scat4, 0)

            def _scat1(j, _):
                v = val_v[pl.ds(j * 16, 16)]
                ix = idx_v[pl.ds(j * 16, 16)]
                plsc.addupdate_scatter(acc2, [lane_off + ix], v)
                return 0

            lax.fori_loop(n4 * 4, nvec, _scat1, 0)

            def _comb(ci, _):
                t = zero16
                for r in range(16):
                    t = t + acc2[pl.ds(r * N_GRAPHS + ci * 16, 16)]
                accv[pl.ds(ci * 16, 16)] = t
                return 0

            lax.fori_loop(0, N_GRAPHS // 16, _comb, 0)
            pltpu.sync_copy(accv, shared.at[s])

        plsc.subcore_barrier()

        @pl.when(on)
        def _final():
            if final:
                cp_p = pltpu.async_copy(
                    prev_hbm.at[pl.ds(s * _COLS, _COLS)], prev_v, sem)
            pltpu.sync_copy(shared, gath)
            if final:
                cp_p.wait()
            for ci in range(_COLS // 16):
                t = jnp.zeros((16,), jnp.float32)
                for r in range(_NS):
                    t = t + gath[r, pl.ds(s * _COLS + ci * 16, 16)]
                if final:
                    t = t + prev_v[pl.ds(ci * 16, 16)]
                    t = t * _STDDEV + _MEAN
                accv[pl.ds(ci * 16, 16)] = t
            pltpu.sync_copy(accv.at[pl.ds(0, _COLS)],
                            out_hbm.at[pl.ds(s * _COLS, _COLS)])

    scratch = [
        pltpu.VMEM((buf,), jnp.float32),
        pltpu.VMEM((buf,), jnp.int32),
        pltpu.VMEM((_NS * N_GRAPHS,), jnp.float32),
        pltpu.VMEM((N_GRAPHS,), jnp.float32),
        pltpu.VMEM_SHARED((_NS, N_GRAPHS), jnp.float32),
        pltpu.VMEM((_NS, N_GRAPHS), jnp.float32),
    ]
    if final:
        scratch.append(pltpu.VMEM((_COLS,), jnp.float32))
    scratch.append(pltpu.SemaphoreType.DMA)

    mesh = plsc.VectorSubcoreMesh(core_axis_name="c", subcore_axis_name="s")
    return pl.kernel(
        body,
        out_type=jax.ShapeDtypeStruct((N_GRAPHS,), jnp.float32),
        mesh=mesh,
        scratch_types=scratch,
        compiler_params=pltpu.CompilerParams(needs_layout_passes=False),
    )


_seg_head = _make_segment_sum(_SPLIT, 0, final=False)
_seg_tail = _make_segment_sum(N_NODES - _SPLIT, _SPLIT, final=True)


@functools.partial(jax.jit)
def kernel(x, W1, b1, W2, batch_idx):
    b1r = b1.reshape(1, HIDDEN)
    idx = batch_idx.astype(jnp.int32)
    o1 = _mlp(x, W1, b1r, W2, 0, _SPLIT)
    o2 = _mlp(x, W1, b1r, W2, _SPLIT, N_NODES - _SPLIT)
    p1 = _seg_head(o1.reshape(_SPLIT), idx)
    agg = _seg_tail(o2.reshape(N_NODES - _SPLIT), idx, p1)
    return agg.reshape(N_GRAPHS, 1)


# speedup vs baseline: 1.0676x; 1.0676x over previous
import functools

import jax
import jax.numpy as jnp
from jax import lax
from jax.experimental import pallas as pl
from jax.experimental.pallas import tpu as pltpu
from jax.experimental.pallas import tpu_sc as plsc
import numpy as np

N_NODES = 100000
NODE_DIM = 512
HIDDEN = 256
N_GRAPHS = 512
_MEAN = 0.0
_STDDEV = 1.0
_LOG2 = float(np.log(2.0))

_SPLIT = 90000
_BLK = 10000


def _mlp_body(x_ref, w1_ref, b1_ref, w2_ref, o_ref):
    h = jnp.dot(x_ref[...], w1_ref[...], preferred_element_type=jnp.float32)
    h = h + b1_ref[...]
    sp = jnp.maximum(h, 0.0) + jnp.log1p(jnp.exp(-jnp.abs(h))) - _LOG2
    o_ref[...] = jnp.dot(sp, w2_ref[...], preferred_element_type=jnp.float32)


def _mlp(x, W1, b1, W2, row0, nrows):
    blk0 = row0 // _BLK
    return pl.pallas_call(
        _mlp_body,
        grid=(nrows // _BLK,),
        in_specs=[
            pl.BlockSpec((_BLK, NODE_DIM), lambda i: (blk0 + i, 0)),
            pl.BlockSpec((NODE_DIM, HIDDEN), lambda i: (0, 0)),
            pl.BlockSpec((1, HIDDEN), lambda i: (0, 0)),
            pl.BlockSpec((HIDDEN, 1), lambda i: (0, 0)),
        ],
        out_specs=pl.BlockSpec((_BLK, 1), lambda i: (i, 0)),
        out_shape=jax.ShapeDtypeStruct((nrows, 1), jnp.float32),
    )(x, W1, b1, W2)


_NS = 16
_COLS = N_GRAPHS // _NS


def _make_segment_sum(n, off, final, has_prev=False):
    chunk = 16 * (n // (16 * _NS))
    tail = n - (_NS - 1) * chunk
    buf = max(chunk, tail)

    def body(*refs):
        if has_prev:
            (vals_hbm, idx_hbm, prev_hbm, out_hbm,
             val_v, idx_v, acc2, accv, shared, gath, prev_v, sem) = refs
        else:
            (vals_hbm, idx_hbm, out_hbm,
             val_v, idx_v, acc2, accv, shared, gath, sem) = refs
        c = lax.axis_index("c")
        s = lax.axis_index("s")
        on = c == 0

        @pl.when(on)
        def _work():
            base = s * chunk

            @pl.when(s < _NS - 1)
            def _full():
                pltpu.async_copy(
                    vals_hbm.at[pl.ds(base, chunk)], val_v.at[pl.ds(0, chunk)], sem)
                pltpu.async_copy(
                    idx_hbm.at[pl.ds(off + base, chunk)], idx_v.at[pl.ds(0, chunk)], sem)

            @pl.when(s == _NS - 1)
            def _tail():
                pltpu.async_copy(
                    vals_hbm.at[pl.ds(base, tail)], val_v.at[pl.ds(0, tail)], sem)
                pltpu.async_copy(
                    idx_hbm.at[pl.ds(off + base, tail)], idx_v.at[pl.ds(0, tail)], sem)

            zero16 = jnp.zeros((16,), jnp.float32)

            def _zero(i, _):
                for k in range(8):
                    acc2[pl.ds((i * 8 + k) * 16, 16)] = zero16
                return 0

            lax.fori_loop(0, (16 * N_GRAPHS) // (16 * 8), _zero, 0)

            @pl.when(s < _NS - 1)
            def _wait_full():
                pltpu.make_async_copy(
                    vals_hbm.at[pl.ds(base, chunk)], val_v.at[pl.ds(0, chunk)], sem).wait()
                pltpu.make_async_copy(
                    idx_hbm.at[pl.ds(off + base, chunk)], idx_v.at[pl.ds(0, chunk)], sem).wait()

            @pl.when(s == _NS - 1)
            def _wait_tail():
                pltpu.make_async_copy(
                    vals_hbm.at[pl.ds(base, tail)], val_v.at[pl.ds(0, tail)], sem).wait()
                pltpu.make_async_copy(
                    idx_hbm.at[pl.ds(off + base, tail)], idx_v.at[pl.ds(0, tail)], sem).wait()

            lane_off = lax.iota(jnp.int32, 16) * N_GRAPHS
            nvec = jnp.where(s == _NS - 1, tail // 16, chunk // 16)
            n4 = nvec // 4

            def _scat4(i, _):
                for k in range(4):
                    j = i * 4 + k
                    v = val_v[pl.ds(j * 16, 16)]
                    ix = idx_v[pl.ds(j * 16, 16)]
                    plsc.addupdate_scatter(acc2, [lane_off + ix], v)
                return 0

            lax.fori_loop(0, n4, _scat4, 0)

            def _scat1(j, _):
                v = val_v[pl.ds(j * 16, 16)]
                ix = idx_v[pl.ds(j * 16, 16)]
                plsc.addupdate_scatter(acc2, [lane_off + ix], v)
                return 0

            lax.fori_loop(n4 * 4, nvec, _scat1, 0)

            def _comb(ci, _):
                t = zero16
                for r in range(16):
                    t = t + acc2[pl.ds(r * N_GRAPHS + ci * 16, 16)]
                accv[pl.ds(ci * 16, 16)] = t
                return 0

            lax.fori_loop(0, N_GRAPHS // 16, _comb, 0)
            pltpu.sync_copy(accv, shared.at[s])

        plsc.subcore_barrier()

        @pl.when(on)
        def _final():
            if has_prev:
                cp_p = pltpu.async_copy(
                    prev_hbm.at[pl.ds(s * _COLS, _COLS)], prev_v, sem)
            pltpu.sync_copy(shared, gath)
            if has_prev:
                cp_p.wait()
            for ci in range(_COLS // 16):
                t = jnp.zeros((16,), jnp.float32)
                for r in range(_NS):
                    t = t + gath[r, pl.ds(s * _COLS + ci * 16, 16)]
                if has_prev:
                    t = t + prev_v[pl.ds(ci * 16, 16)]
                if final:
                    t = t * _STDDEV + _MEAN
                accv[pl.ds(ci * 16, 16)] = t
            pltpu.sync_copy(accv.at[pl.ds(0, _COLS)],
                            out_hbm.at[pl.ds(s * _COLS, _COLS)])

    scratch = [
        pltpu.VMEM((buf,), jnp.float32),
        pltpu.VMEM((buf,), jnp.int32),
        pltpu.VMEM((_NS * N_GRAPHS,), jnp.float32),
        pltpu.VMEM((N_GRAPHS,), jnp.float32),
        pltpu.VMEM_SHARED((_NS, N_GRAPHS), jnp.float32),
        pltpu.VMEM((_NS, N_GRAPHS), jnp.float32),
    ]
    if has_prev:
        scratch.append(pltpu.VMEM((_COLS,), jnp.float32))
    scratch.append(pltpu.SemaphoreType.DMA)

    mesh = plsc.VectorSubcoreMesh(core_axis_name="c", subcore_axis_name="s")
    return pl.kernel(
        body,
        out_type=jax.ShapeDtypeStruct((N_GRAPHS,), jnp.float32),
        mesh=mesh,
        scratch_types=scratch,
        compiler_params=pltpu.CompilerParams(needs_layout_passes=False),
    )


_seg_all = _make_segment_sum(N_NODES, 0, final=True)


@functools.partial(jax.jit)
def kernel(x, W1, b1, W2, batch_idx):
    b1r = b1.reshape(1, HIDDEN)
    idx = batch_idx.astype(jnp.int32)
    o = _mlp(x, W1, b1r, W2, 0, N_NODES)
    agg = _seg_all(o.reshape(N_NODES), idx)
    return agg.reshape(N_GRAPHS, 1)


# speedup vs baseline: 1.0849x; 1.0162x over previous
import functools

import jax
import jax.numpy as jnp
from jax import lax
from jax.experimental import pallas as pl
from jax.experimental.pallas import tpu as pltpu
from jax.experimental.pallas import tpu_sc as plsc
import numpy as np

N_NODES = 100000
NODE_DIM = 512
HIDDEN = 256
N_GRAPHS = 512
_MEAN = 0.0
_STDDEV = 1.0
_LOG2 = float(np.log(2.0))

_SPLIT = 90000
_BLK = 10000


def _mlp_body(x_ref, w1_ref, b1_ref, w2_ref, o_ref):
    h = jnp.dot(x_ref[...], w1_ref[...], preferred_element_type=jnp.float32)
    h = h + b1_ref[...]
    sp = jnp.maximum(h, 0.0) + jnp.log1p(jnp.exp(-jnp.abs(h))) - _LOG2
    o_ref[...] = jnp.dot(sp, w2_ref[...], preferred_element_type=jnp.float32)


def _mlp(x, W1, b1, W2, row0, nrows):
    blk0 = row0 // _BLK
    return pl.pallas_call(
        _mlp_body,
        grid=(nrows // _BLK,),
        in_specs=[
            pl.BlockSpec((_BLK, NODE_DIM), lambda i: (blk0 + i, 0)),
            pl.BlockSpec((NODE_DIM, HIDDEN), lambda i: (0, 0)),
            pl.BlockSpec((1, HIDDEN), lambda i: (0, 0)),
            pl.BlockSpec((HIDDEN, 1), lambda i: (0, 0)),
        ],
        out_specs=pl.BlockSpec((_BLK, 1), lambda i: (i, 0)),
        out_shape=jax.ShapeDtypeStruct((nrows, 1), jnp.float32),
    )(x, W1, b1, W2)


_NS = 16
_COLS = N_GRAPHS // _NS


def _make_segment_sum(n, off, final, has_prev=False):
    chunk = 16 * (n // (16 * _NS))
    tail = n - (_NS - 1) * chunk
    buf = max(chunk, tail)

    def body(*refs):
        if has_prev:
            (vals_hbm, idx_hbm, prev_hbm, out_hbm,
             val_v, idx_v, acc2, accv, shared, gath, prev_v, sem) = refs
        else:
            (vals_hbm, idx_hbm, out_hbm,
             val_v, idx_v, acc2, accv, shared, gath, sem) = refs
        c = lax.axis_index("c")
        s = lax.axis_index("s")
        on = c == 0

        @pl.when(on)
        def _work():
            base = s * chunk

            @pl.when(s < _NS - 1)
            def _full():
                pltpu.async_copy(
                    vals_hbm.at[pl.ds(base, chunk)], val_v.at[pl.ds(0, chunk)], sem)
                pltpu.async_copy(
                    idx_hbm.at[pl.ds(off + base, chunk)], idx_v.at[pl.ds(0, chunk)], sem)

            @pl.when(s == _NS - 1)
            def _tail():
                pltpu.async_copy(
                    vals_hbm.at[pl.ds(base, tail)], val_v.at[pl.ds(0, tail)], sem)
                pltpu.async_copy(
                    idx_hbm.at[pl.ds(off + base, tail)], idx_v.at[pl.ds(0, tail)], sem)

            zero16 = jnp.zeros((16,), jnp.float32)

            @plsc.parallel_loop(0, (16 * N_GRAPHS) // 16, unroll=8)
            def _zero(i):
                acc2[pl.ds(i * 16, 16)] = zero16

            @pl.when(s < _NS - 1)
            def _wait_full():
                pltpu.make_async_copy(
                    vals_hbm.at[pl.ds(base, chunk)], val_v.at[pl.ds(0, chunk)], sem).wait()
                pltpu.make_async_copy(
                    idx_hbm.at[pl.ds(off + base, chunk)], idx_v.at[pl.ds(0, chunk)], sem).wait()

            @pl.when(s == _NS - 1)
            def _wait_tail():
                pltpu.make_async_copy(
                    vals_hbm.at[pl.ds(base, tail)], val_v.at[pl.ds(0, tail)], sem).wait()
                pltpu.make_async_copy(
                    idx_hbm.at[pl.ds(off + base, tail)], idx_v.at[pl.ds(0, tail)], sem).wait()

            lane_off = lax.iota(jnp.int32, 16) * N_GRAPHS
            nvec = jnp.where(s == _NS - 1, tail // 16, chunk // 16)

            @plsc.parallel_loop(0, nvec, unroll=4)
            def _scat(j):
                v = val_v[pl.ds(j * 16, 16)]
                ix = idx_v[pl.ds(j * 16, 16)]
                plsc.addupdate_scatter(acc2, [lane_off + ix], v)

            @plsc.parallel_loop(0, N_GRAPHS // 16, unroll=2)
            def _comb(ci):
                t = zero16
                for r in range(16):
                    t = t + acc2[pl.ds(r * N_GRAPHS + ci * 16, 16)]
                accv[pl.ds(ci * 16, 16)] = t
            pltpu.sync_copy(accv, shared.at[s])

        plsc.subcore_barrier()

        @pl.when(on)
        def _final():
            if has_prev:
                cp_p = pltpu.async_copy(
                    prev_hbm.at[pl.ds(s * _COLS, _COLS)], prev_v, sem)
            pltpu.sync_copy(shared, gath)
            if has_prev:
                cp_p.wait()
            for ci in range(_COLS // 16):
                t = jnp.zeros((16,), jnp.float32)
                for r in range(_NS):
                    t = t + gath[r, pl.ds(s * _COLS + ci * 16, 16)]
                if has_prev:
                    t = t + prev_v[pl.ds(ci * 16, 16)]
                if final:
                    t = t * _STDDEV + _MEAN
                accv[pl.ds(ci * 16, 16)] = t
            pltpu.sync_copy(accv.at[pl.ds(0, _COLS)],
                            out_hbm.at[pl.ds(s * _COLS, _COLS)])

    scratch = [
        pltpu.VMEM((buf,), jnp.float32),
        pltpu.VMEM((buf,), jnp.int32),
        pltpu.VMEM((_NS * N_GRAPHS,), jnp.float32),
        pltpu.VMEM((N_GRAPHS,), jnp.float32),
        pltpu.VMEM_SHARED((_NS, N_GRAPHS), jnp.float32),
        pltpu.VMEM((_NS, N_GRAPHS), jnp.float32),
    ]
    if has_prev:
        scratch.append(pltpu.VMEM((_COLS,), jnp.float32))
    scratch.append(pltpu.SemaphoreType.DMA)

    mesh = plsc.VectorSubcoreMesh(core_axis_name="c", subcore_axis_name="s")
    return pl.kernel(
        body,
        out_type=jax.ShapeDtypeStruct((N_GRAPHS,), jnp.float32),
        mesh=mesh,
        scratch_types=scratch,
        compiler_params=pltpu.CompilerParams(needs_layout_passes=False),
    )


_seg_all = _make_segment_sum(N_NODES, 0, final=True)


@functools.partial(jax.jit)
def kernel(x, W1, b1, W2, batch_idx):
    b1r = b1.reshape(1, HIDDEN)
    idx = batch_idx.astype(jnp.int32)
    o = _mlp(x, W1, b1r, W2, 0, N_NODES)
    agg = _seg_all(o.reshape(N_NODES), idx)
    return agg.reshape(N_GRAPHS, 1)


# speedup vs baseline: 1.0913x; 1.0059x over previous
import functools

import jax
import jax.numpy as jnp
from jax import lax
from jax.experimental import pallas as pl
from jax.experimental.pallas import tpu as pltpu
from jax.experimental.pallas import tpu_sc as plsc
import numpy as np

N_NODES = 100000
NODE_DIM = 512
HIDDEN = 256
N_GRAPHS = 512
_MEAN = 0.0
_STDDEV = 1.0
_LOG2 = float(np.log(2.0))

_SPLIT = 90000
_BLK = 10000


_KH = NODE_DIM // 2


def _mlp_body(xl_ref, xr_ref, w1_ref, b1_ref, w2_ref, o_ref):
    h = jnp.dot(xl_ref[...], w1_ref[pl.ds(0, _KH), :],
                preferred_element_type=jnp.float32)
    h = h + jnp.dot(xr_ref[...], w1_ref[pl.ds(_KH, _KH), :],
                    preferred_element_type=jnp.float32)
    h = h + b1_ref[...]
    sp = jnp.maximum(h, 0.0) + jnp.log1p(jnp.exp(-jnp.abs(h))) - _LOG2
    o_ref[...] = jnp.dot(sp, w2_ref[...], preferred_element_type=jnp.float32)


def _mlp(x, W1, b1, W2, row0, nrows):
    blk0 = row0 // _BLK
    return pl.pallas_call(
        _mlp_body,
        grid=(nrows // _BLK,),
        in_specs=[
            pl.BlockSpec((_BLK, _KH), lambda i: (blk0 + i, 0)),
            pl.BlockSpec((_BLK, _KH), lambda i: (blk0 + i, 1)),
            pl.BlockSpec((NODE_DIM, HIDDEN), lambda i: (0, 0)),
            pl.BlockSpec((1, HIDDEN), lambda i: (0, 0)),
            pl.BlockSpec((HIDDEN, 1), lambda i: (0, 0)),
        ],
        out_specs=pl.BlockSpec((_BLK, 1), lambda i: (i, 0)),
        out_shape=jax.ShapeDtypeStruct((nrows, 1), jnp.float32),
    )(x, x, W1, b1, W2)


_NS = 16
_COLS = N_GRAPHS // _NS


def _make_segment_sum(n, off, final, has_prev=False):
    chunk = 16 * (n // (16 * _NS))
    tail = n - (_NS - 1) * chunk
    buf = max(chunk, tail)

    def body(*refs):
        if has_prev:
            (vals_hbm, idx_hbm, prev_hbm, out_hbm,
             val_v, idx_v, acc2, accv, shared, gath, prev_v, sem) = refs
        else:
            (vals_hbm, idx_hbm, out_hbm,
             val_v, idx_v, acc2, accv, shared, gath, sem) = refs
        c = lax.axis_index("c")
        s = lax.axis_index("s")
        on = c == 0

        @pl.when(on)
        def _work():
            base = s * chunk

            @pl.when(s < _NS - 1)
            def _full():
                pltpu.async_copy(
                    vals_hbm.at[pl.ds(base, chunk)], val_v.at[pl.ds(0, chunk)], sem)
                pltpu.async_copy(
                    idx_hbm.at[pl.ds(off + base, chunk)], idx_v.at[pl.ds(0, chunk)], sem)

            @pl.when(s == _NS - 1)
            def _tail():
                pltpu.async_copy(
                    vals_hbm.at[pl.ds(base, tail)], val_v.at[pl.ds(0, tail)], sem)
                pltpu.async_copy(
                    idx_hbm.at[pl.ds(off + base, tail)], idx_v.at[pl.ds(0, tail)], sem)

            zero16 = jnp.zeros((16,), jnp.float32)

            @plsc.parallel_loop(0, (16 * N_GRAPHS) // 16, unroll=8)
            def _zero(i):
                acc2[pl.ds(i * 16, 16)] = zero16

            @pl.when(s < _NS - 1)
            def _wait_full():
                pltpu.make_async_copy(
                    vals_hbm.at[pl.ds(base, chunk)], val_v.at[pl.ds(0, chunk)], sem).wait()
                pltpu.make_async_copy(
                    idx_hbm.at[pl.ds(off + base, chunk)], idx_v.at[pl.ds(0, chunk)], sem).wait()

            @pl.when(s == _NS - 1)
            def _wait_tail():
                pltpu.make_async_copy(
                    vals_hbm.at[pl.ds(base, tail)], val_v.at[pl.ds(0, tail)], sem).wait()
                pltpu.make_async_copy(
                    idx_hbm.at[pl.ds(off + base, tail)], idx_v.at[pl.ds(0, tail)], sem).wait()

            lane_off = lax.iota(jnp.int32, 16) * N_GRAPHS
            nvec = jnp.where(s == _NS - 1, tail // 16, chunk // 16)

            @plsc.parallel_loop(0, nvec, unroll=4)
            def _scat(j):
                v = val_v[pl.ds(j * 16, 16)]
                ix = idx_v[pl.ds(j * 16, 16)]
                plsc.addupdate_scatter(acc2, [lane_off + ix], v)

            @plsc.parallel_loop(0, N_GRAPHS // 16, unroll=2)
            def _comb(ci):
                t = zero16
                for r in range(16):
                    t = t + acc2[pl.ds(r * N_GRAPHS + ci * 16, 16)]
                accv[pl.ds(ci * 16, 16)] = t
            pltpu.sync_copy(accv, shared.at[s])

        plsc.subcore_barrier()

        @pl.when(on)
        def _final():
            if has_prev:
                cp_p = pltpu.async_copy(
                    prev_hbm.at[pl.ds(s * _COLS, _COLS)], prev_v, sem)
            pltpu.sync_copy(shared, gath)
            if has_prev:
                cp_p.wait()
            for ci in range(_COLS // 16):
                t = jnp.zeros((16,), jnp.float32)
                for r in range(_NS):
                    t = t + gath[r, pl.ds(s * _COLS + ci * 16, 16)]
                if has_prev:
                    t = t + prev_v[pl.ds(ci * 16, 16)]
                if final:
                    t = t * _STDDEV + _MEAN
                accv[pl.ds(ci * 16, 16)] = t
            pltpu.sync_copy(accv.at[pl.ds(0, _COLS)],
                            out_hbm.at[pl.ds(s * _COLS, _COLS)])

    scratch = [
        pltpu.VMEM((buf,), jnp.float32),
        pltpu.VMEM((buf,), jnp.int32),
        pltpu.VMEM((_NS * N_GRAPHS,), jnp.float32),
        pltpu.VMEM((N_GRAPHS,), jnp.float32),
        pltpu.VMEM_SHARED((_NS, N_GRAPHS), jnp.float32),
        pltpu.VMEM((_NS, N_GRAPHS), jnp.float32),
    ]
    if has_prev:
        scratch.append(pltpu.VMEM((_COLS,), jnp.float32))
    scratch.append(pltpu.SemaphoreType.DMA)

    mesh = plsc.VectorSubcoreMesh(core_axis_name="c", subcore_axis_name="s")
    return pl.kernel(
        body,
        out_type=jax.ShapeDtypeStruct((N_GRAPHS,), jnp.float32),
        mesh=mesh,
        scratch_types=scratch,
        compiler_params=pltpu.CompilerParams(needs_layout_passes=False),
    )


_seg_all = _make_segment_sum(N_NODES, 0, final=True)


@functools.partial(jax.jit)
def kernel(x, W1, b1, W2, batch_idx):
    b1r = b1.reshape(1, HIDDEN)
    idx = batch_idx.astype(jnp.int32)
    o = _mlp(x, W1, b1r, W2, 0, N_NODES)
    agg = _seg_all(o.reshape(N_NODES), idx)
    return agg.reshape(N_GRAPHS, 1)
